# baseline (speedup 1.0000x reference)
import functools

import jax
import jax.numpy as jnp
from jax import lax
from jax.experimental import pallas as pl
from jax.experimental.pallas import tpu as pltpu
from jax.experimental.pallas import tpu_sc as plsc

_NC = 2
_NS = 16
_NW = _NC * _NS
_CHUNK = 128


def _sc_edge_agg(x, src3, dstf, zeros_rows, n_pad, k):
    N, D = x.shape
    z_rows = n_pad // _NS
    mesh = plsc.VectorSubcoreMesh(core_axis_name="c", subcore_axis_name="s")

    @functools.partial(
        pl.kernel,
        out_type=[jax.ShapeDtypeStruct((n_pad, D), jnp.float32),
                  jax.ShapeDtypeStruct((n_pad, D), jnp.float32)],
        mesh=mesh,
        scratch_types=[
            pltpu.VMEM((k, _CHUNK), jnp.int32),
            pltpu.VMEM((_CHUNK,), jnp.int32),
            pltpu.VMEM((_CHUNK,), jnp.int32),
            pltpu.VMEM((_CHUNK, D), jnp.float32),
            pltpu.VMEM((_CHUNK, D), jnp.float32),
            pltpu.VMEM_SHARED((n_pad, D), jnp.float32),
            pltpu.SemaphoreType.DMA,
            pltpu.SemaphoreType.DMA,
            pltpu.SemaphoreType.DMA,
            pltpu.SemaphoreType.DMA,
        ],
    )
    def agg_kernel(x_hbm, src_hbm, dst_hbm, zer_hbm, out0_hbm, out1_hbm,
                   src_v, db0, db1, buf0, buf1, agg_sh, sg0, sg1, sd0, sd1):
        c = lax.axis_index("c")
        s = lax.axis_index("s")
        wid = s * _NC + c
        base = wid * k * _CHUNK
        pltpu.sync_copy(zer_hbm, agg_sh.at[pl.ds(s * z_rows, z_rows)])
        pltpu.sync_copy(src_hbm.at[wid], src_v)
        plsc.subcore_barrier()

        def body(i2, carry):
            j = 2 * i2
            pltpu.async_copy(x_hbm.at[src_v.at[j]], buf0, sg0)
            pltpu.async_copy(x_hbm.at[src_v.at[j + 1]], buf1, sg1)
            pltpu.async_copy(
                dst_hbm.at[pl.ds(base + j * _CHUNK, _CHUNK)], db0, sd0)
            pltpu.async_copy(
                dst_hbm.at[pl.ds(base + (j + 1) * _CHUNK, _CHUNK)], db1, sd1)
            pltpu.make_async_copy(x_hbm.at[src_v.at[j]], buf0, sg0).wait()
            pltpu.make_async_copy(
                dst_hbm.at[pl.ds(base, _CHUNK)], db0, sd0).wait()
            pltpu.sync_copy(buf0, agg_sh.at[db0], add=True)
            pltpu.make_async_copy(
                x_hbm.at[src_v.at[j + 1]], buf1, sg1).wait()
            pltpu.make_async_copy(
                dst_hbm.at[pl.ds(base, _CHUNK)], db1, sd1).wait()
            pltpu.sync_copy(buf1, agg_sh.at[db1], add=True)
            return carry

        lax.fori_loop(0, k // 2, body, 0)
        plsc.subcore_barrier()

        @pl.when(c == 0)
        def _():
            pltpu.sync_copy(agg_sh.at[pl.ds(s * z_rows, z_rows)],
                            out0_hbm.at[pl.ds(s * z_rows, z_rows)])

        @pl.when(c == 1)
        def _():
            pltpu.sync_copy(agg_sh.at[pl.ds(s * z_rows, z_rows)],
                            out1_hbm.at[pl.ds(s * z_rows, z_rows)])

    return agg_kernel(x, src3, dstf, zeros_rows)


def _tc_head(x, agg0, agg1, batch3, W1, b1r, W2, b2r, scale_r, bnb_r,
             fc1_W, fc1b_r, fc2_Wp, fc2b_r, nb, B, interpret=False):
    N, D = x.shape
    DIM = W1.shape[1]
    G = 128

    def body(x_ref, a0_ref, a1_ref, bt_ref, W1_ref, b1_ref, W2_ref, b2_ref,
             sc_ref, bnb_ref, f1W_ref, f1b_ref, f2W_ref, f2b_ref,
             out_ref, acc_ref):
        i = pl.program_id(0)
        h = x_ref[...] + a0_ref[...] + a1_ref[...]
        z = jnp.maximum(
            jnp.dot(h, W1_ref[...], preferred_element_type=jnp.float32)
            + b1_ref[...], 0.0)
        z = jnp.dot(z, W2_ref[...], preferred_element_type=jnp.float32) \
            + b2_ref[...]
        z = jnp.maximum(z, 0.0) * sc_ref[...] + bnb_ref[...]
        iota_g = lax.broadcasted_iota(jnp.int32, (G, B), 0)
        p_t = (iota_g == bt_ref[0, 0, :].reshape(1, B)).astype(jnp.float32)
        pooled = jnp.dot(p_t, z, preferred_element_type=jnp.float32,
                         precision=lax.Precision.HIGHEST)

        @pl.when(i == 0)
        def _():
            acc_ref[...] = jnp.zeros_like(acc_ref)

        acc_ref[...] += pooled

        @pl.when(i == nb - 1)
        def _():
            g = jnp.maximum(
                jnp.dot(acc_ref[...], f1W_ref[...],
                        preferred_element_type=jnp.float32) + f1b_ref[...],
                0.0)
            out_ref[...] = jnp.dot(
                g, f2W_ref[...], preferred_element_type=jnp.float32) \
                + f2b_ref[...]

    return pl.pallas_call(
        body,
        grid=(nb,),
        in_specs=[
            pl.BlockSpec((B, D), lambda i: (i, 0)),
            pl.BlockSpec((B, D), lambda i: (i, 0)),
            pl.BlockSpec((B, D), lambda i: (i, 0)),
            pl.BlockSpec((1, 1, B), lambda i: (i, 0, 0)),
            pl.BlockSpec((D, DIM), lambda i: (0, 0)),
            pl.BlockSpec((1, DIM), lambda i: (0, 0)),
            pl.BlockSpec((DIM, DIM), lambda i: (0, 0)),
            pl.BlockSpec((1, DIM), lambda i: (0, 0)),
            pl.BlockSpec((1, DIM), lambda i: (0, 0)),
            pl.BlockSpec((1, DIM), lambda i: (0, 0)),
            pl.BlockSpec((DIM, DIM), lambda i: (0, 0)),
            pl.BlockSpec((1, DIM), lambda i: (0, 0)),
            pl.BlockSpec((DIM, 128), lambda i: (0, 0)),
            pl.BlockSpec((1, 128), lambda i: (0, 0)),
        ],
        out_specs=pl.BlockSpec((G, 128), lambda i: (0, 0)),
        out_shape=jax.ShapeDtypeStruct((G, 128), jnp.float32),
        scratch_shapes=[pltpu.VMEM((G, DIM), jnp.float32)],
        interpret=interpret,
    )(x, agg0, agg1, batch3, W1, b1r, W2, b2r, scale_r, bnb_r,
      fc1_W, fc1b_r, fc2_Wp, fc2b_r)


def kernel(x, edge_index, batch, W1, b1, W2, b2, bn_g, bn_b,
           fc1_W, fc1_b, fc2_W, fc2_b):
    N, D = x.shape
    DIM = W1.shape[1]
    E = edge_index.shape[1]

    per_w = -(-E // _NW)
    k = -(-per_w // _CHUNK)
    k += k % 2
    e_pad = _NW * k * _CHUNK
    n_pad = (N // 128 + 1) * 128
    src = edge_index[0]
    dst = edge_index[1]
    pad = e_pad - E
    if pad:
        src = jnp.concatenate([src, jnp.zeros((pad,), jnp.int32)])
        dst = jnp.concatenate([dst, jnp.full((pad,), N, jnp.int32)])
    src3 = src.reshape(_NW, k, _CHUNK)
    dstf = dst.reshape(-1)
    zeros_rows = jnp.zeros((n_pad // _NS, D), jnp.float32)
    agg0, agg1 = _sc_edge_agg(x, src3, dstf, zeros_rows, n_pad, k)

    B = 400
    nb = N // B
    batch3 = batch.reshape(nb, 1, B)
    b1r = b1.reshape(1, DIM)
    b2r = b2.reshape(1, DIM)
    scale_r = (bn_g * (1.0 / jnp.sqrt(1.0 + 1e-5))).reshape(1, DIM)
    bnb_r = bn_b.reshape(1, DIM)
    fc1b_r = fc1_b.reshape(1, DIM)
    fc2_Wp = jnp.pad(fc2_W, ((0, 0), (0, 127)))
    fc2b_r = jnp.broadcast_to(fc2_b.reshape(1, 1), (1, 128))
    out = _tc_head(x, agg0, agg1, batch3, W1, b1r, W2, b2r, scale_r, bnb_r,
                   fc1_W, fc1b_r, fc2_Wp, fc2b_r, nb, B)
    return out[:, :1]


# speedup vs baseline: 1.5033x; 1.5033x over previous
import functools

import jax
import jax.numpy as jnp
from jax import lax
from jax.experimental import pallas as pl
from jax.experimental.pallas import tpu as pltpu
from jax.experimental.pallas import tpu_sc as plsc

_NC = 2
_NS = 16
_NW = _NC * _NS
_CHUNK = 128


def _sc_edge_agg(x, src3, dst3, zeros_rows, n_pad, k):
    N, D = x.shape
    z_rows = n_pad // _NS
    mesh = plsc.VectorSubcoreMesh(core_axis_name="c", subcore_axis_name="s")

    @functools.partial(
        pl.kernel,
        out_type=[jax.ShapeDtypeStruct((n_pad, D), jnp.float32),
                  jax.ShapeDtypeStruct((n_pad, D), jnp.float32)],
        mesh=mesh,
        scratch_types=[
            pltpu.VMEM((k, _CHUNK), jnp.int32),
            pltpu.VMEM((k, _CHUNK), jnp.int32),
            pltpu.VMEM((_CHUNK, D), jnp.float32),
            pltpu.VMEM_SHARED((n_pad, D), jnp.float32),
            pltpu.SemaphoreType.DMA,
        ],
    )
    def agg_kernel(x_hbm, src_hbm, dst_hbm, zer_hbm, out0_hbm, out1_hbm,
                   src_v, dst_v, rows_v, agg_sh, sem):
        c = lax.axis_index("c")
        s = lax.axis_index("s")
        wid = s * _NC + c
        pltpu.sync_copy(zer_hbm, agg_sh.at[pl.ds(s * z_rows, z_rows)])
        pltpu.sync_copy(src_hbm.at[wid], src_v)
        pltpu.sync_copy(dst_hbm.at[wid], dst_v)
        plsc.subcore_barrier()

        def body(j, carry):
            pltpu.async_copy(x_hbm.at[src_v.at[j]], rows_v, sem).wait()
            pltpu.sync_copy(rows_v, agg_sh.at[dst_v.at[j]], add=True)
            return carry

        lax.fori_loop(0, k, body, 0)
        plsc.subcore_barrier()

        @pl.when(c == 0)
        def _():
            pltpu.sync_copy(agg_sh.at[pl.ds(s * z_rows, z_rows)],
                            out0_hbm.at[pl.ds(s * z_rows, z_rows)])

        @pl.when(c == 1)
        def _():
            pltpu.sync_copy(agg_sh.at[pl.ds(s * z_rows, z_rows)],
                            out1_hbm.at[pl.ds(s * z_rows, z_rows)])

    return agg_kernel(x, src3, dst3, zeros_rows)


def _tc_head(x, agg0, agg1, batch3, W1, b1r, W2, b2r, scale_r, bnb_r,
             fc1_W, fc1b_r, fc2_Wp, fc2b_r, nb, B, interpret=False):
    N, D = x.shape
    DIM = W1.shape[1]
    G = 128

    def body(x_ref, a0_ref, a1_ref, bt_ref, W1_ref, b1_ref, W2_ref, b2_ref,
             sc_ref, bnb_ref, f1W_ref, f1b_ref, f2W_ref, f2b_ref,
             out_ref, acc_ref):
        i = pl.program_id(0)
        h = x_ref[...] + a0_ref[...] + a1_ref[...]
        z = jnp.maximum(
            jnp.dot(h, W1_ref[...], preferred_element_type=jnp.float32)
            + b1_ref[...], 0.0)
        z = jnp.dot(z, W2_ref[...], preferred_element_type=jnp.float32) \
            + b2_ref[...]
        z = jnp.maximum(z, 0.0) * sc_ref[...] + bnb_ref[...]
        iota_g = lax.broadcasted_iota(jnp.int32, (G, B), 0)
        p_t = (iota_g == bt_ref[0, 0, :].reshape(1, B)).astype(jnp.float32)
        pooled = jnp.dot(p_t, z, preferred_element_type=jnp.float32,
                         precision=lax.Precision.HIGHEST)

        @pl.when(i == 0)
        def _():
            acc_ref[...] = jnp.zeros_like(acc_ref)

        acc_ref[...] += pooled

        @pl.when(i == nb - 1)
        def _():
            g = jnp.maximum(
                jnp.dot(acc_ref[...], f1W_ref[...],
                        preferred_element_type=jnp.float32) + f1b_ref[...],
                0.0)
            out_ref[...] = jnp.dot(
                g, f2W_ref[...], preferred_element_type=jnp.float32) \
                + f2b_ref[...]

    return pl.pallas_call(
        body,
        grid=(nb,),
        in_specs=[
            pl.BlockSpec((B, D), lambda i: (i, 0)),
            pl.BlockSpec((B, D), lambda i: (i, 0)),
            pl.BlockSpec((B, D), lambda i: (i, 0)),
            pl.BlockSpec((1, 1, B), lambda i: (i, 0, 0)),
            pl.BlockSpec((D, DIM), lambda i: (0, 0)),
            pl.BlockSpec((1, DIM), lambda i: (0, 0)),
            pl.BlockSpec((DIM, DIM), lambda i: (0, 0)),
            pl.BlockSpec((1, DIM), lambda i: (0, 0)),
            pl.BlockSpec((1, DIM), lambda i: (0, 0)),
            pl.BlockSpec((1, DIM), lambda i: (0, 0)),
            pl.BlockSpec((DIM, DIM), lambda i: (0, 0)),
            pl.BlockSpec((1, DIM), lambda i: (0, 0)),
            pl.BlockSpec((DIM, 128), lambda i: (0, 0)),
            pl.BlockSpec((1, 128), lambda i: (0, 0)),
        ],
        out_specs=pl.BlockSpec((G, 128), lambda i: (0, 0)),
        out_shape=jax.ShapeDtypeStruct((G, 128), jnp.float32),
        scratch_shapes=[pltpu.VMEM((G, DIM), jnp.float32)],
        interpret=interpret,
    )(x, agg0, agg1, batch3, W1, b1r, W2, b2r, scale_r, bnb_r,
      fc1_W, fc1b_r, fc2_Wp, fc2b_r)


def kernel(x, edge_index, batch, W1, b1, W2, b2, bn_g, bn_b,
           fc1_W, fc1_b, fc2_W, fc2_b):
    N, D = x.shape
    DIM = W1.shape[1]
    E = edge_index.shape[1]

    per_w = -(-E // _NW)
    k = -(-per_w // _CHUNK)
    e_pad = _NW * k * _CHUNK
    n_pad = (N // 128 + 1) * 128
    src = edge_index[0]
    dst = edge_index[1]
    pad = e_pad - E
    if pad:
        src = jnp.concatenate([src, jnp.zeros((pad,), jnp.int32)])
        dst = jnp.concatenate([dst, jnp.full((pad,), N, jnp.int32)])
    src3 = src.reshape(_NW, k, _CHUNK)
    dst3 = dst.reshape(_NW, k, _CHUNK)
    zeros_rows = jnp.zeros((n_pad // _NS, D), jnp.float32)
    agg0, agg1 = _sc_edge_agg(x, src3, dst3, zeros_rows, n_pad, k)

    B = 2000
    nb = N // B
    batch3 = batch.reshape(nb, 1, B)
    b1r = b1.reshape(1, DIM)
    b2r = b2.reshape(1, DIM)
    scale_r = (bn_g * (1.0 / jnp.sqrt(1.0 + 1e-5))).reshape(1, DIM)
    bnb_r = bn_b.reshape(1, DIM)
    fc1b_r = fc1_b.reshape(1, DIM)
    fc2_Wp = jnp.pad(fc2_W, ((0, 0), (0, 127)))
    fc2b_r = jnp.broadcast_to(fc2_b.reshape(1, 1), (1, 128))
    out = _tc_head(x, agg0, agg1, batch3, W1, b1r, W2, b2r, scale_r, bnb_r,
                   fc1_W, fc1b_r, fc2_Wp, fc2b_r, nb, B)
    return out[:, :1]


# speedup vs baseline: 1.5268x; 1.0156x over previous
import functools

import jax
import jax.numpy as jnp
from jax import lax
from jax.experimental import pallas as pl
from jax.experimental.pallas import tpu as pltpu
from jax.experimental.pallas import tpu_sc as plsc

_NC = 2
_NS = 16
_NW = _NC * _NS
_CHUNK = 128


def _sc_edge_agg(x, src3, dst3, zeros_rows, n_pad, k):
    N, D = x.shape
    z_rows = n_pad // _NS
    mesh = plsc.VectorSubcoreMesh(core_axis_name="c", subcore_axis_name="s")

    @functools.partial(
        pl.kernel,
        out_type=[jax.ShapeDtypeStruct((n_pad, D), jnp.float32),
                  jax.ShapeDtypeStruct((n_pad, D), jnp.float32)],
        mesh=mesh,
        scratch_types=[
            pltpu.VMEM((k, _CHUNK), jnp.int32),
            pltpu.VMEM((k, _CHUNK), jnp.int32),
            pltpu.VMEM((_CHUNK, D), jnp.float32),
            pltpu.VMEM_SHARED((n_pad, D), jnp.float32),
            pltpu.SemaphoreType.DMA,
        ],
    )
    def agg_kernel(x_hbm, src_hbm, dst_hbm, zer_hbm, out0_hbm, out1_hbm,
                   src_v, dst_v, rows_v, agg_sh, sem):
        c = lax.axis_index("c")
        s = lax.axis_index("s")
        wid = s * _NC + c
        zero16 = jnp.zeros((16,), jnp.float32)

        def zfill(r, carry):
            for t2 in range(D // 16):
                rows_v[r, pl.ds(t2 * 16, 16)] = zero16
            return carry

        lax.fori_loop(0, _CHUNK, zfill, 0)
        for r0 in range(0, z_rows, _CHUNK):
            ln = min(_CHUNK, z_rows - r0)
            pltpu.sync_copy(rows_v.at[pl.ds(0, ln)],
                            agg_sh.at[pl.ds(s * z_rows + r0, ln)])
        pltpu.sync_copy(src_hbm.at[wid], src_v)
        pltpu.sync_copy(dst_hbm.at[wid], dst_v)
        plsc.subcore_barrier()

        def body(j, carry):
            pltpu.async_copy(x_hbm.at[src_v.at[j]], rows_v, sem).wait()
            pltpu.sync_copy(rows_v, agg_sh.at[dst_v.at[j]], add=True)
            return carry

        lax.fori_loop(0, k, body, 0)
        plsc.subcore_barrier()

        @pl.when(c == 0)
        def _():
            pltpu.sync_copy(agg_sh.at[pl.ds(s * z_rows, z_rows)],
                            out0_hbm.at[pl.ds(s * z_rows, z_rows)])

        @pl.when(c == 1)
        def _():
            pltpu.sync_copy(agg_sh.at[pl.ds(s * z_rows, z_rows)],
                            out1_hbm.at[pl.ds(s * z_rows, z_rows)])

    return agg_kernel(x, src3, dst3, zeros_rows)


def _tc_head(x, agg0, agg1, batch3, W1, b1r, W2, b2r, scale_r, bnb_r,
             fc1_W, fc1b_r, fc2_Wp, fc2b_r, nb, B, interpret=False):
    N, D = x.shape
    DIM = W1.shape[1]
    G = 128

    def body(x_ref, a0_ref, a1_ref, bt_ref, W1_ref, b1_ref, W2_ref, b2_ref,
             sc_ref, bnb_ref, f1W_ref, f1b_ref, f2W_ref, f2b_ref,
             out_ref, acc_ref):
        i = pl.program_id(0)
        h = x_ref[...] + a0_ref[...] + a1_ref[...]
        z = jnp.maximum(
            jnp.dot(h, W1_ref[...], preferred_element_type=jnp.float32)
            + b1_ref[...], 0.0)
        z = jnp.dot(z, W2_ref[...], preferred_element_type=jnp.float32) \
            + b2_ref[...]
        z = jnp.maximum(z, 0.0) * sc_ref[...] + bnb_ref[...]
        iota_g = lax.broadcasted_iota(jnp.int32, (G, B), 0)
        p_t = (iota_g == bt_ref[0, 0, :].reshape(1, B)).astype(jnp.float32)
        pooled = jnp.dot(p_t, z, preferred_element_type=jnp.float32,
                         precision=lax.Precision.HIGHEST)

        @pl.when(i == 0)
        def _():
            acc_ref[...] = jnp.zeros_like(acc_ref)

        acc_ref[...] += pooled

        @pl.when(i == nb - 1)
        def _():
            g = jnp.maximum(
                jnp.dot(acc_ref[...], f1W_ref[...],
                        preferred_element_type=jnp.float32) + f1b_ref[...],
                0.0)
            out_ref[...] = jnp.dot(
                g, f2W_ref[...], preferred_element_type=jnp.float32) \
                + f2b_ref[...]

    return pl.pallas_call(
        body,
        grid=(nb,),
        in_specs=[
            pl.BlockSpec((B, D), lambda i: (i, 0)),
            pl.BlockSpec((B, D), lambda i: (i, 0)),
            pl.BlockSpec((B, D), lambda i: (i, 0)),
            pl.BlockSpec((1, 1, B), lambda i: (i, 0, 0)),
            pl.BlockSpec((D, DIM), lambda i: (0, 0)),
            pl.BlockSpec((1, DIM), lambda i: (0, 0)),
            pl.BlockSpec((DIM, DIM), lambda i: (0, 0)),
            pl.BlockSpec((1, DIM), lambda i: (0, 0)),
            pl.BlockSpec((1, DIM), lambda i: (0, 0)),
            pl.BlockSpec((1, DIM), lambda i: (0, 0)),
            pl.BlockSpec((DIM, DIM), lambda i: (0, 0)),
            pl.BlockSpec((1, DIM), lambda i: (0, 0)),
            pl.BlockSpec((DIM, 128), lambda i: (0, 0)),
            pl.BlockSpec((1, 128), lambda i: (0, 0)),
        ],
        out_specs=pl.BlockSpec((G, 128), lambda i: (0, 0)),
        out_shape=jax.ShapeDtypeStruct((G, 128), jnp.float32),
        scratch_shapes=[pltpu.VMEM((G, DIM), jnp.float32)],
        interpret=interpret,
    )(x, agg0, agg1, batch3, W1, b1r, W2, b2r, scale_r, bnb_r,
      fc1_W, fc1b_r, fc2_Wp, fc2b_r)


def kernel(x, edge_index, batch, W1, b1, W2, b2, bn_g, bn_b,
           fc1_W, fc1_b, fc2_W, fc2_b):
    N, D = x.shape
    DIM = W1.shape[1]
    E = edge_index.shape[1]

    per_w = -(-E // _NW)
    k = -(-per_w // _CHUNK)
    e_pad = _NW * k * _CHUNK
    n_pad = (N // 128 + 1) * 128
    src = edge_index[0]
    dst = edge_index[1]
    pad = e_pad - E
    if pad:
        src = jnp.concatenate([src, jnp.zeros((pad,), jnp.int32)])
        dst = jnp.concatenate([dst, jnp.full((pad,), N, jnp.int32)])
    src3 = src.reshape(_NW, k, _CHUNK)
    dst3 = dst.reshape(_NW, k, _CHUNK)
    zeros_rows = jnp.zeros((n_pad // _NS, D), jnp.float32)
    agg0, agg1 = _sc_edge_agg(x, src3, dst3, zeros_rows, n_pad, k)

    B = 2000
    nb = N // B
    batch3 = batch.reshape(nb, 1, B)
    b1r = b1.reshape(1, DIM)
    b2r = b2.reshape(1, DIM)
    scale_r = (bn_g * (1.0 / jnp.sqrt(1.0 + 1e-5))).reshape(1, DIM)
    bnb_r = bn_b.reshape(1, DIM)
    fc1b_r = fc1_b.reshape(1, DIM)
    fc2_Wp = jnp.pad(fc2_W, ((0, 0), (0, 127)))
    fc2b_r = jnp.broadcast_to(fc2_b.reshape(1, 1), (1, 128))
    out = _tc_head(x, agg0, agg1, batch3, W1, b1r, W2, b2r, scale_r, bnb_r,
                   fc1_W, fc1b_r, fc2_Wp, fc2b_r, nb, B)
    return out[:, :1]
